# Initial kernel scaffold; baseline (speedup 1.0000x reference)
import jax
import jax.numpy as jnp
from jax.experimental import pallas as pl


def kernel(feat, fij, rij, edge_index, W_in2f, Wf1, bf1, Wf2, bf2, Wm1, bm1, Wm2, bm2):
    raise NotImplementedError("write your pallas kernel here")


# speedup vs baseline: 1.4092x; 1.4092x over previous
import functools

import jax
import jax.numpy as jnp
from jax import lax
from jax.experimental import pallas as pl
from jax.experimental.pallas import tpu as pltpu, tpu_sc as plsc

N_NODES = 10000
N_EDGES = 320000
D = 128
G = 50
CUTOFF = 5.0

NC = 2
NS = 16
NW = NC * NS
EDGES_PER_TILE = N_EDGES // NW
CHUNK = 80
N_CHUNKS = EDGES_PER_TILE // CHUNK
N_PAD = 10240
ROWS_PER_TILE = N_PAD // NS

E_BLK = 2560
N_BLK = 2000


def _ssp(x):
    return jnp.logaddexp(x, 0.0) - jnp.log(2.0)


def _he_body(fij_ref, rij_ref, wf1_ref, bf1_ref, wf2_ref, bf2_ref, he_ref):
    h1 = _ssp(
        jnp.dot(fij_ref[...], wf1_ref[...], preferred_element_type=jnp.float32)
        + bf1_ref[...]
    )
    he = (
        jnp.dot(h1, wf2_ref[...], preferred_element_type=jnp.float32)
        + bf2_ref[...]
    )
    r = rij_ref[...]
    c = 0.5 * (jnp.cos(r * (jnp.pi / CUTOFF)) + 1.0) * (r < CUTOFF).astype(jnp.float32)
    he_ref[...] = he * c


def _compute_he(fij, rij2d, Wf1, bf1, Wf2, bf2):
    n_blocks = N_EDGES // E_BLK
    return pl.pallas_call(
        _he_body,
        grid=(n_blocks,),
        in_specs=[
            pl.BlockSpec((E_BLK, G), lambda i: (i, 0)),
            pl.BlockSpec((E_BLK, 1), lambda i: (i, 0)),
            pl.BlockSpec((G, D), lambda i: (0, 0)),
            pl.BlockSpec((1, D), lambda i: (0, 0)),
            pl.BlockSpec((D, D), lambda i: (0, 0)),
            pl.BlockSpec((1, D), lambda i: (0, 0)),
        ],
        out_specs=pl.BlockSpec((E_BLK, D), lambda i: (i, 0)),
        out_shape=jax.ShapeDtypeStruct((N_EDGES, D), jnp.float32),
    )(fij, rij2d, Wf1, bf1, Wf2, bf2)


def _hv_body(feat_ref, w_ref, hv_ref):
    hv_ref[...] = jnp.dot(
        feat_ref[...], w_ref[...], preferred_element_type=jnp.float32
    )


def _compute_hv(feat, W_in2f):
    return pl.pallas_call(
        _hv_body,
        grid=(5,),
        in_specs=[
            pl.BlockSpec((N_NODES // 5, D), lambda i: (i, 0)),
            pl.BlockSpec((D, D), lambda i: (0, 0)),
        ],
        out_specs=pl.BlockSpec((N_NODES // 5, D), lambda i: (i, 0)),
        out_shape=jax.ShapeDtypeStruct((N_NODES, D), jnp.float32),
    )(feat, W_in2f)


def _sc_body(hv_hbm, he_hbm, src_hbm, dst_hbm, zeros_hbm, out_hbm,
             m_sh, src_v, dst_v, rows_v, he_v, gsem):
    c = lax.axis_index("c")
    s = lax.axis_index("s")
    wid = s * NC + c

    pltpu.sync_copy(zeros_hbm, m_sh.at[pl.ds(s * ROWS_PER_TILE, ROWS_PER_TILE)])
    plsc.subcore_barrier()

    def chunk_body(j, carry):
        base = wid * EDGES_PER_TILE + j * CHUNK
        pltpu.sync_copy(src_hbm.at[pl.ds(base, CHUNK)], src_v)
        pltpu.sync_copy(dst_hbm.at[pl.ds(base, CHUNK)], dst_v)
        pltpu.async_copy(hv_hbm.at[src_v], rows_v, gsem).wait()
        pltpu.sync_copy(he_hbm.at[pl.ds(base, CHUNK)], he_v)

        def row_body(r, carry2):
            for k in range(D // 16):
                sl = pl.ds(k * 16, 16)
                rows_v[r, sl] = rows_v[r, sl] * he_v[r, sl]
            return carry2

        lax.fori_loop(0, CHUNK, row_body, 0, unroll=False)
        pltpu.sync_copy(rows_v, m_sh.at[dst_v], add=True)
        return carry

    lax.fori_loop(0, N_CHUNKS, chunk_body, 0, unroll=False)

    plsc.subcore_barrier()
    pltpu.sync_copy(
        m_sh.at[pl.ds(s * ROWS_PER_TILE, ROWS_PER_TILE)],
        out_hbm.at[pl.ds(c * N_PAD + s * ROWS_PER_TILE, ROWS_PER_TILE)],
    )


def _sc_aggregate(hv, he, src, dst, zeros):
    mesh = plsc.VectorSubcoreMesh(core_axis_name="c", subcore_axis_name="s")
    kfn = functools.partial(
        pl.kernel,
        out_type=jax.ShapeDtypeStruct((NC * N_PAD, D), jnp.float32),
        mesh=mesh,
        scratch_types=[
            pltpu.VMEM_SHARED((N_PAD, D), jnp.float32),
            pltpu.VMEM((CHUNK,), jnp.int32),
            pltpu.VMEM((CHUNK,), jnp.int32),
            pltpu.VMEM((CHUNK, D), jnp.float32),
            pltpu.VMEM((CHUNK, D), jnp.float32),
            pltpu.SemaphoreType.DMA,
        ],
    )(_sc_body)
    return kfn(hv, he, src, dst, zeros)


def _out_body(p_ref, wm1_ref, bm1_ref, wm2_ref, bm2_ref, out_ref):
    m = p_ref[0] + p_ref[1]
    h = jnp.dot(m, wm1_ref[...], preferred_element_type=jnp.float32) + bm1_ref[...]
    h = h * jax.nn.sigmoid(h)
    out_ref[...] = (
        jnp.dot(h, wm2_ref[...], preferred_element_type=jnp.float32) + bm2_ref[...]
    )


def _compute_out(partials, Wm1, bm1, Wm2, bm2):
    n_blocks = N_NODES // N_BLK
    return pl.pallas_call(
        _out_body,
        grid=(n_blocks,),
        in_specs=[
            pl.BlockSpec((2, N_BLK, D), lambda i: (0, i, 0)),
            pl.BlockSpec((D, D), lambda i: (0, 0)),
            pl.BlockSpec((1, D), lambda i: (0, 0)),
            pl.BlockSpec((D, D), lambda i: (0, 0)),
            pl.BlockSpec((1, D), lambda i: (0, 0)),
        ],
        out_specs=pl.BlockSpec((N_BLK, D), lambda i: (i, 0)),
        out_shape=jax.ShapeDtypeStruct((N_NODES, D), jnp.float32),
    )(partials, Wm1, bm1, Wm2, bm2)


def kernel(feat, fij, rij, edge_index, W_in2f, Wf1, bf1, Wf2, bf2, Wm1, bm1, Wm2, bm2):
    src = edge_index[0]
    dst = edge_index[1]
    rij2d = rij.reshape(N_EDGES, 1)
    zeros = jnp.zeros((ROWS_PER_TILE, D), jnp.float32)

    he = _compute_he(fij, rij2d, Wf1, bf1.reshape(1, D), Wf2, bf2.reshape(1, D))
    hv = _compute_hv(feat, W_in2f)
    partials = _sc_aggregate(hv, he, src, dst, zeros).reshape(NC, N_PAD, D)
    return _compute_out(partials, Wm1, bm1.reshape(1, D), Wm2, bm2.reshape(1, D))


# speedup vs baseline: 3.7008x; 2.6261x over previous
import functools

import jax
import jax.numpy as jnp
from jax import lax
from jax.experimental import pallas as pl
from jax.experimental.pallas import tpu as pltpu, tpu_sc as plsc

N_NODES = 10000
N_EDGES = 320000
D = 128
G = 50
CUTOFF = 5.0

NC = 2
NS = 16
NW = NC * NS
EDGES_PER_TILE = N_EDGES // NW
CHUNK = 40
N_CHUNKS = EDGES_PER_TILE // CHUNK
N_PAD = 10240
ROWS_PER_TILE = N_PAD // NS

E_BLK = 2560
N_BLK = 2000

_SIN_COEFS = (1.5707963e0, -6.4596409e-1, 7.9692594e-2,
              -4.6816369e-3, 1.6023519e-4, -3.4252394e-6)


def _c_body(r_ref, c_ref):
    r = r_ref[...]
    u = r * (2.0 / CUTOFF) - 1.0
    u2 = u * u
    p = _SIN_COEFS[-1]
    for coef in _SIN_COEFS[-2::-1]:
        p = p * u2 + coef
    cval = 0.5 - 0.5 * (p * u)
    c_ref[...] = jnp.where(r < CUTOFF, cval, 0.0)


def _compute_c(rij_packed):
    return pl.pallas_call(
        _c_body,
        out_shape=jax.ShapeDtypeStruct(rij_packed.shape, jnp.float32),
    )(rij_packed)


def _he_body(fijT_ref, c_ref, wf1_ref, bf1_ref, wf2_ref, bf2_ref, he_ref):
    h1 = lax.dot_general(
        fijT_ref[...], wf1_ref[...],
        dimension_numbers=(((0,), (0,)), ((), ())),
        preferred_element_type=jnp.float32,
    ) + bf1_ref[...]
    sp = jnp.maximum(h1, 0.0) + jnp.log1p(jnp.exp(-jnp.abs(h1))) - 0.6931471805599453
    he = (
        jnp.dot(sp, wf2_ref[...], preferred_element_type=jnp.float32)
        + bf2_ref[...]
    )
    he_ref[...] = he * c_ref[...]


def _compute_he(fijT, c_col, Wf1, bf1, bf2s, Wf2):
    n_blocks = N_EDGES // E_BLK
    return pl.pallas_call(
        _he_body,
        grid=(n_blocks,),
        in_specs=[
            pl.BlockSpec((G, E_BLK), lambda i: (0, i)),
            pl.BlockSpec((E_BLK, 1), lambda i: (i, 0)),
            pl.BlockSpec((G, D), lambda i: (0, 0)),
            pl.BlockSpec((1, D), lambda i: (0, 0)),
            pl.BlockSpec((D, D), lambda i: (0, 0)),
            pl.BlockSpec((1, D), lambda i: (0, 0)),
        ],
        out_specs=pl.BlockSpec((E_BLK, D), lambda i: (i, 0)),
        out_shape=jax.ShapeDtypeStruct((N_EDGES, D), jnp.float32),
    )(fijT, c_col, Wf1, bf1, Wf2, bf2s)


def _hv_body(feat_ref, w_ref, hv_ref):
    hv_ref[...] = jnp.dot(
        feat_ref[...], w_ref[...], preferred_element_type=jnp.float32
    )


def _compute_hv(feat, W_in2f):
    return pl.pallas_call(
        _hv_body,
        grid=(5,),
        in_specs=[
            pl.BlockSpec((N_NODES // 5, D), lambda i: (i, 0)),
            pl.BlockSpec((D, D), lambda i: (0, 0)),
        ],
        out_specs=pl.BlockSpec((N_NODES // 5, D), lambda i: (i, 0)),
        out_shape=jax.ShapeDtypeStruct((N_NODES, D), jnp.float32),
    )(feat, W_in2f)


def _sc_body(hv_hbm, he_hbm, src_hbm, dst_hbm, zeros_hbm, out_hbm,
             m_sh, src_t, rows_b, he_b, dst_b, slm):
    c = lax.axis_index("c")
    s = lax.axis_index("s")
    wid = s * NC + c
    ebase = wid * EDGES_PER_TILE

    pltpu.sync_copy(zeros_hbm, m_sh.at[pl.ds(s * ROWS_PER_TILE, ROWS_PER_TILE)])
    pltpu.sync_copy(src_hbm.at[pl.ds(ebase, EDGES_PER_TILE)], src_t)
    plsc.subcore_barrier()

    def issue(j, b):
        pltpu.async_copy(
            hv_hbm.at[src_t.at[pl.ds(j * CHUNK, CHUNK)]], rows_b[b], slm[b])
        pltpu.async_copy(
            he_hbm.at[pl.ds(ebase + j * CHUNK, CHUNK)], he_b[b], slm[b])
        pltpu.async_copy(
            dst_hbm.at[pl.ds(ebase + j * CHUNK, CHUNK)], dst_b[b], slm[b])

    def process(j, b):
        pltpu.make_async_copy(
            hv_hbm.at[src_t.at[pl.ds(j * CHUNK, CHUNK)]], rows_b[b], slm[b]).wait()
        pltpu.make_async_copy(
            he_hbm.at[pl.ds(ebase + j * CHUNK, CHUNK)], he_b[b], slm[b]).wait()
        pltpu.make_async_copy(
            dst_hbm.at[pl.ds(ebase + j * CHUNK, CHUNK)], dst_b[b], slm[b]).wait()

        rows, he = rows_b[b], he_b[b]

        def row_body(r, carry):
            for k in range(D // 16):
                sl = pl.ds(k * 16, 16)
                rows[r, sl] = rows[r, sl] * he[r, sl]
            return carry

        lax.fori_loop(0, CHUNK, row_body, 0, unroll=False)
        pltpu.sync_copy(rows, m_sh.at[dst_b[b]], add=True)

    issue(0, 0)
    issue(1, 1)

    def pair_step(p, carry):
        process(2 * p, 0)

        @pl.when(p < N_CHUNKS // 2 - 1)
        def _():
            issue(2 * p + 2, 0)

        process(2 * p + 1, 1)

        @pl.when(p < N_CHUNKS // 2 - 1)
        def _():
            issue(2 * p + 3, 1)

        return carry

    lax.fori_loop(0, N_CHUNKS // 2, pair_step, 0, unroll=False)

    plsc.subcore_barrier()
    pltpu.sync_copy(
        m_sh.at[pl.ds(s * ROWS_PER_TILE, ROWS_PER_TILE)],
        out_hbm.at[pl.ds(c * N_PAD + s * ROWS_PER_TILE, ROWS_PER_TILE)],
    )


def _sc_aggregate(hv, he, src, dst, zeros):
    mesh = plsc.VectorSubcoreMesh(core_axis_name="c", subcore_axis_name="s")
    kfn = functools.partial(
        pl.kernel,
        out_type=jax.ShapeDtypeStruct((NC * N_PAD, D), jnp.float32),
        mesh=mesh,
        scratch_types=[
            pltpu.VMEM_SHARED((N_PAD, D), jnp.float32),
            pltpu.VMEM((EDGES_PER_TILE,), jnp.int32),
            [pltpu.VMEM((CHUNK, D), jnp.float32) for _ in range(2)],
            [pltpu.VMEM((CHUNK, D), jnp.float32) for _ in range(2)],
            [pltpu.VMEM((CHUNK,), jnp.int32) for _ in range(2)],
            [pltpu.SemaphoreType.DMA for _ in range(2)],
        ],
    )(_sc_body)
    return kfn(hv, he, src, dst, zeros)


def _out_body(p_ref, wm1_ref, bm1_ref, wm2_ref, bm2_ref, out_ref):
    m = p_ref[0] + p_ref[1]
    h = jnp.dot(m, wm1_ref[...], preferred_element_type=jnp.float32) + bm1_ref[...]
    h = h * jax.nn.sigmoid(h)
    out_ref[...] = (
        jnp.dot(h, wm2_ref[...], preferred_element_type=jnp.float32) + bm2_ref[...]
    )


def _compute_out(partials, Wm1, bm1, Wm2, bm2):
    n_blocks = N_NODES // N_BLK
    return pl.pallas_call(
        _out_body,
        grid=(n_blocks,),
        in_specs=[
            pl.BlockSpec((2, N_BLK, D), lambda i: (0, i, 0)),
            pl.BlockSpec((D, D), lambda i: (0, 0)),
            pl.BlockSpec((1, D), lambda i: (0, 0)),
            pl.BlockSpec((D, D), lambda i: (0, 0)),
            pl.BlockSpec((1, D), lambda i: (0, 0)),
        ],
        out_specs=pl.BlockSpec((N_BLK, D), lambda i: (i, 0)),
        out_shape=jax.ShapeDtypeStruct((N_NODES, D), jnp.float32),
    )(partials, Wm1, bm1, Wm2, bm2)


def kernel(feat, fij, rij, edge_index, W_in2f, Wf1, bf1, Wf2, bf2, Wm1, bm1, Wm2, bm2):
    src = edge_index[0]
    dst = edge_index[1]
    zeros = jnp.zeros((ROWS_PER_TILE, D), jnp.float32)
    bf2s = bf2.reshape(1, D)

    c_col = _compute_c(rij.reshape(N_EDGES // D, D)).reshape(N_EDGES, 1)
    he = _compute_he(fij.T, c_col, Wf1, bf1.reshape(1, D), bf2s, Wf2)
    hv = _compute_hv(feat, W_in2f)
    partials = _sc_aggregate(hv, he, src, dst, zeros).reshape(NC, N_PAD, D)
    return _compute_out(partials, Wm1, bm1.reshape(1, D), Wm2, bm2.reshape(1, D))


# speedup vs baseline: 3.8534x; 1.0412x over previous
import functools

import jax
import jax.numpy as jnp
from jax import lax
from jax.experimental import pallas as pl
from jax.experimental.pallas import tpu as pltpu, tpu_sc as plsc

N_NODES = 10000
N_EDGES = 320000
D = 128
G = 50
CUTOFF = 5.0

NC = 2
NS = 16
NW = NC * NS
N_EDGES_H = N_EDGES // 2
EDGES_PER_TILE = N_EDGES_H // NW
CHUNK = 40
N_CHUNKS = EDGES_PER_TILE // CHUNK
N_PAD = 10240
ROWS_PER_TILE = N_PAD // NS

E_BLK = 3200
N_BLK = 2000

_SIN_COEFS = (1.5707963e0, -6.4596409e-1, 7.9692594e-2,
              -4.6816369e-3, 1.6023519e-4, -3.4252394e-6)


def _c_body(r_ref, c_ref):
    r = r_ref[...]
    u = r * (2.0 / CUTOFF) - 1.0
    u2 = u * u
    p = _SIN_COEFS[-1]
    for coef in _SIN_COEFS[-2::-1]:
        p = p * u2 + coef
    cval = 0.5 - 0.5 * (p * u)
    c_ref[...] = jnp.where(r < CUTOFF, cval, 0.0)


def _compute_c(rij_packed):
    return pl.pallas_call(
        _c_body,
        out_shape=jax.ShapeDtypeStruct(rij_packed.shape, jnp.float32),
    )(rij_packed)


def _he_body(fijT_ref, c_ref, wf1_ref, bf1_ref, wf2_ref, bf2_ref, he_ref):
    h1 = lax.dot_general(
        fijT_ref[...], wf1_ref[...],
        dimension_numbers=(((0,), (0,)), ((), ())),
        preferred_element_type=jnp.float32,
    ) + bf1_ref[...]
    sp = jnp.maximum(h1, 0.0) + jnp.log1p(jnp.exp(-jnp.abs(h1))) - 0.6931471805599453
    he = (
        jnp.dot(sp, wf2_ref[...], preferred_element_type=jnp.float32)
        + bf2_ref[...]
    )
    he_ref[...] = he * c_ref[...]


def _compute_he(fijT, c_col, Wf1, bf1, bf2s, Wf2):
    n_blocks = N_EDGES_H // E_BLK
    return pl.pallas_call(
        _he_body,
        grid=(n_blocks,),
        in_specs=[
            pl.BlockSpec((G, E_BLK), lambda i: (0, i)),
            pl.BlockSpec((E_BLK, 1), lambda i: (i, 0)),
            pl.BlockSpec((G, D), lambda i: (0, 0)),
            pl.BlockSpec((1, D), lambda i: (0, 0)),
            pl.BlockSpec((D, D), lambda i: (0, 0)),
            pl.BlockSpec((1, D), lambda i: (0, 0)),
        ],
        out_specs=pl.BlockSpec((E_BLK, D), lambda i: (i, 0)),
        out_shape=jax.ShapeDtypeStruct((N_EDGES_H, D), jnp.float32),
    )(fijT, c_col, Wf1, bf1, Wf2, bf2s)


def _hv_body(feat_ref, w_ref, hv_ref):
    hv_ref[...] = jnp.dot(
        feat_ref[...], w_ref[...], preferred_element_type=jnp.float32
    )


def _compute_hv(feat, W_in2f):
    return pl.pallas_call(
        _hv_body,
        grid=(5,),
        in_specs=[
            pl.BlockSpec((N_NODES // 5, D), lambda i: (i, 0)),
            pl.BlockSpec((D, D), lambda i: (0, 0)),
        ],
        out_specs=pl.BlockSpec((N_NODES // 5, D), lambda i: (i, 0)),
        out_shape=jax.ShapeDtypeStruct((N_NODES, D), jnp.float32),
    )(feat, W_in2f)


def _sc_body(hv_hbm, he_hbm, src_hbm, dst_hbm, zeros_hbm, out_hbm,
             m_sh, src_t, rows_b, he_b, dst_b, slm):
    c = lax.axis_index("c")
    s = lax.axis_index("s")
    wid = s * NC + c
    ebase = wid * EDGES_PER_TILE

    pltpu.sync_copy(zeros_hbm, m_sh.at[pl.ds(s * ROWS_PER_TILE, ROWS_PER_TILE)])
    pltpu.sync_copy(src_hbm.at[pl.ds(ebase, EDGES_PER_TILE)], src_t)
    plsc.subcore_barrier()

    def issue(j, b):
        pltpu.async_copy(
            hv_hbm.at[src_t.at[pl.ds(j * CHUNK, CHUNK)]], rows_b[b], slm[b])
        pltpu.async_copy(
            he_hbm.at[pl.ds(ebase + j * CHUNK, CHUNK)], he_b[b], slm[b])
        pltpu.async_copy(
            dst_hbm.at[pl.ds(ebase + j * CHUNK, CHUNK)], dst_b[b], slm[b])

    def process(j, b):
        pltpu.make_async_copy(
            hv_hbm.at[src_t.at[pl.ds(j * CHUNK, CHUNK)]], rows_b[b], slm[b]).wait()
        pltpu.make_async_copy(
            he_hbm.at[pl.ds(ebase + j * CHUNK, CHUNK)], he_b[b], slm[b]).wait()
        pltpu.make_async_copy(
            dst_hbm.at[pl.ds(ebase + j * CHUNK, CHUNK)], dst_b[b], slm[b]).wait()

        rows, he = rows_b[b], he_b[b]

        def row_body(r, carry):
            for k in range(D // 16):
                sl = pl.ds(k * 16, 16)
                rows[r, sl] = rows[r, sl] * he[r, sl]
            return carry

        lax.fori_loop(0, CHUNK, row_body, 0, unroll=False)
        pltpu.sync_copy(rows, m_sh.at[dst_b[b]], add=True)

    issue(0, 0)
    issue(1, 1)

    def pair_step(p, carry):
        process(2 * p, 0)
        issue(2 * p + 2, 0)
        process(2 * p + 1, 1)

        @pl.when(p < N_CHUNKS // 2 - 1)
        def _():
            issue(2 * p + 3, 1)

        return carry

    lax.fori_loop(0, N_CHUNKS // 2, pair_step, 0, unroll=False)
    process(N_CHUNKS - 1, 0)

    plsc.subcore_barrier()
    pltpu.sync_copy(
        m_sh.at[pl.ds(s * ROWS_PER_TILE, ROWS_PER_TILE)],
        out_hbm.at[pl.ds(c * N_PAD + s * ROWS_PER_TILE, ROWS_PER_TILE)],
    )


def _sc_aggregate(hv, he, src, dst, zeros):
    mesh = plsc.VectorSubcoreMesh(core_axis_name="c", subcore_axis_name="s")
    kfn = functools.partial(
        pl.kernel,
        out_type=jax.ShapeDtypeStruct((NC * N_PAD, D), jnp.float32),
        mesh=mesh,
        scratch_types=[
            pltpu.VMEM_SHARED((N_PAD, D), jnp.float32),
            pltpu.VMEM((EDGES_PER_TILE,), jnp.int32),
            [pltpu.VMEM((CHUNK, D), jnp.float32) for _ in range(2)],
            [pltpu.VMEM((CHUNK, D), jnp.float32) for _ in range(2)],
            [pltpu.VMEM((CHUNK,), jnp.int32) for _ in range(2)],
            [pltpu.SemaphoreType.DMA for _ in range(2)],
        ],
    )(_sc_body)
    return kfn(hv, he, src, dst, zeros)


def _out_body(p_ref, wm1_ref, bm1_ref, wm2_ref, bm2_ref, out_ref):
    m = (p_ref[0] + p_ref[1]) + (p_ref[2] + p_ref[3])
    h = jnp.dot(m, wm1_ref[...], preferred_element_type=jnp.float32) + bm1_ref[...]
    h = h * jax.nn.sigmoid(h)
    out_ref[...] = (
        jnp.dot(h, wm2_ref[...], preferred_element_type=jnp.float32) + bm2_ref[...]
    )


def _compute_out(partials, Wm1, bm1, Wm2, bm2):
    n_blocks = N_NODES // N_BLK
    return pl.pallas_call(
        _out_body,
        grid=(n_blocks,),
        in_specs=[
            pl.BlockSpec((4, N_BLK, D), lambda i: (0, i, 0)),
            pl.BlockSpec((D, D), lambda i: (0, 0)),
            pl.BlockSpec((1, D), lambda i: (0, 0)),
            pl.BlockSpec((D, D), lambda i: (0, 0)),
            pl.BlockSpec((1, D), lambda i: (0, 0)),
        ],
        out_specs=pl.BlockSpec((N_BLK, D), lambda i: (i, 0)),
        out_shape=jax.ShapeDtypeStruct((N_NODES, D), jnp.float32),
    )(partials, Wm1, bm1, Wm2, bm2)


def kernel(feat, fij, rij, edge_index, W_in2f, Wf1, bf1, Wf2, bf2, Wm1, bm1, Wm2, bm2):
    src = edge_index[0]
    dst = edge_index[1]
    zeros = jnp.zeros((ROWS_PER_TILE, D), jnp.float32)
    bf2s = bf2.reshape(1, D)

    hv = _compute_hv(feat, W_in2f)
    fijT = fij.T
    parts = []
    for h in range(2):
        sl = slice(h * N_EDGES_H, (h + 1) * N_EDGES_H)
        c_col = _compute_c(
            rij[sl].reshape(N_EDGES_H // D, D)).reshape(N_EDGES_H, 1)
        he = _compute_he(fijT[:, sl], c_col, Wf1, bf1.reshape(1, D), bf2s, Wf2)
        parts.append(_sc_aggregate(hv, he, src[sl], dst[sl], zeros))
    partials = jnp.stack(
        [parts[0].reshape(NC, N_PAD, D), parts[1].reshape(NC, N_PAD, D)],
        axis=0).reshape(2 * NC, N_PAD, D)
    return _compute_out(partials, Wm1, bm1.reshape(1, D), Wm2, bm2.reshape(1, D))
